# baseline (speedup 1.0000x reference)
import jax
import jax.numpy as jnp
import numpy as np
from jax.experimental import pallas as pl
from jax.experimental.pallas import tpu as pltpu

N = 4 * 512 * 512
R, C = 8192, 128
K = N // 4 + 4096
KR = K // C
GRID = 4
BR = KR // GRID
PAD = np.float32(2.0)
ONE_SIGMA = np.float32(1.15)


def _part(gv, gi, pv, pi, base_mask):
    flag1 = gv / gi
    flag2 = gi / gv
    target = jnp.where(flag1 >= ONE_SIGMA, jnp.float32(1.0), jnp.float32(0.0))
    target = jnp.where(flag2 > ONE_SIGMA, jnp.float32(-1.0), target)
    diff = pv - pi
    nz = jnp.logical_and(target != 0.0, base_mask)
    z = jnp.logical_and(target == 0.0, base_mask)
    log_terms = jnp.log(1.0 + jnp.exp(-target * diff))
    s_log = jnp.sum(jnp.where(nz, log_terms, 0.0))
    c_nz = jnp.sum(nz.astype(jnp.float32))
    s_sq = jnp.sum(jnp.where(z, diff * diff, 0.0))
    c_z = jnp.sum(z.astype(jnp.float32))
    return s_log, c_nz, s_sq, c_z


def _rowshift2(x):
    return jnp.concatenate([x[:, 2:], x[:, :2]], axis=1)


def _thre_kernel(d_ref, out_ref):
    d = d_ref[...]
    valid = d > 0.0
    bits = pltpu.bitcast(d, jnp.int32)
    n_pos = jnp.sum(valid.astype(jnp.int32))

    q_index = jnp.float32(0.75) * (n_pos - 1)
    n_pos_f = (n_pos - 1).astype(jnp.float32)
    low = jnp.clip(jnp.floor(q_index), 0, n_pos_f)
    high = jnp.clip(jnp.ceil(q_index), 0, n_pos_f)
    high_weight = q_index - low
    low_weight = jnp.float32(1) - high_weight
    low_i = low.astype(jnp.int32)
    high_i = high.astype(jnp.int32)

    def body(_, lohi):
        lo, hi = lohi
        mid = (lo + hi) // 2
        cnt = jnp.sum(jnp.logical_and(valid, bits <= mid).astype(jnp.int32))
        take = cnt >= low_i + 1
        return (jnp.where(take, lo, mid + 1), jnp.where(take, mid, hi))

    lo_pat, _ = jax.lax.fori_loop(0, 30, body, (jnp.int32(0),
                                                jnp.int32(0x3F7FFFFF)))

    low_value = jnp.min(jnp.where(jnp.logical_and(valid, bits >= lo_pat),
                                  d, jnp.float32(2.0)))
    c_eq = jnp.sum(jnp.logical_and(valid, bits <= lo_pat).astype(jnp.int32))
    next_value = jnp.min(jnp.where(jnp.logical_and(valid, bits > lo_pat),
                                   d, jnp.float32(2.0)))
    high_value = jnp.where(high_i <= c_eq - 1, low_value, next_value)
    thre = low_value * low_weight + high_value * high_weight

    n_a = jnp.sum(jnp.logical_and(valid, d <= thre).astype(jnp.int32))
    n_b = jnp.sum((d > thre).astype(jnp.int32))

    slot = jax.lax.broadcasted_iota(jnp.int32, (1, 4), 1)
    vals = [thre, n_a.astype(jnp.float32), n_b.astype(jnp.float32),
            n_pos.astype(jnp.float32)]
    out_ref[...] = sum(v * (slot == k).astype(jnp.float32)
                       for k, v in enumerate(vals))


def _loss_kernel(s_ref, gn_ref, gf_ref, pn_ref, pf_ref, out_ref):
    step = pl.program_id(0)
    half = s_ref[0, 0]
    ms = s_ref[0, 1]

    gn = gn_ref[...]
    gf = gf_ref[...]
    pn = pn_ref[...]
    pf = pf_ref[...]

    row = jax.lax.broadcasted_iota(jnp.int32, (BR, C), 0) + step * BR
    lane = jax.lax.broadcasted_iota(jnp.int32, (BR, C), 1)
    p = row * C + lane
    mask12 = jnp.logical_and((p % 4) == 0, (p // 4) < half)
    mask3 = jnp.logical_and((p % 2) == 1, ((p - 1) // 2) < ms)

    r1 = _part(gn, _rowshift2(gn), pn, _rowshift2(pn), mask12)
    r2 = _part(gf, _rowshift2(gf), pf, _rowshift2(pf), mask12)
    r3 = _part(gn, gf, pn, pf, mask3)

    sums = [r1[k] + r2[k] + r3[k] for k in range(4)]
    slot = jax.lax.broadcasted_iota(jnp.int32, (1, 4), 1)
    vec = sum(sums[k] * (slot == k).astype(jnp.float32) for k in range(4))

    @pl.when(step == 0)
    def _init():
        out_ref[...] = vec

    @pl.when(step != 0)
    def _acc():
        out_ref[...] = out_ref[...] + vec


def kernel(pred_depth, gt_depth):
    pred = pred_depth.reshape(-1)
    depth = gt_depth.reshape(-1)
    valid = depth > 0

    stats = pl.pallas_call(
        _thre_kernel,
        out_shape=jax.ShapeDtypeStruct((1, 4), jnp.float32),
    )(depth.reshape(R, C))
    thre = stats[0, 0]
    n_a = stats[0, 1].astype(jnp.int32)
    n_b = stats[0, 2].astype(jnp.int32)
    n_pos = stats[0, 3].astype(jnp.int32)

    mask_A = jnp.logical_and(depth <= thre, valid)
    mask_B = depth > thre
    gn = -jax.lax.top_k(jnp.where(valid, -depth, -PAD), K)[0]
    gf = jax.lax.top_k(jnp.where(mask_B, depth, -PAD), K)[0]
    pn = -jax.lax.top_k(jnp.where(mask_A, -pred, -PAD), K)[0]
    pf = jax.lax.top_k(jnp.where(mask_B, pred, -PAD), K)[0]
    m = jnp.minimum(n_a, n_b)
    scalars = jnp.stack([m // 4, m // 2]).reshape(1, 2).astype(jnp.int32)

    sums = pl.pallas_call(
        _loss_kernel,
        grid=(GRID,),
        in_specs=[
            pl.BlockSpec((1, 2), lambda i: (0, 0)),
            pl.BlockSpec((BR, C), lambda i: (i, 0)),
            pl.BlockSpec((BR, C), lambda i: (i, 0)),
            pl.BlockSpec((BR, C), lambda i: (i, 0)),
            pl.BlockSpec((BR, C), lambda i: (i, 0)),
        ],
        out_specs=pl.BlockSpec((1, 4), lambda i: (0, 0)),
        out_shape=jax.ShapeDtypeStruct((1, 4), jnp.float32),
    )(scalars, gn.reshape(KR, C), gf.reshape(KR, C),
      pn.reshape(KR, C), pf.reshape(KR, C))

    log_loss = sums[0, 0] / sums[0, 1]
    squared_loss = sums[0, 2] / sums[0, 3]
    loss = jnp.where(jnp.isnan(log_loss), squared_loss,
                     jnp.where(jnp.isnan(squared_loss), log_loss,
                               log_loss + squared_loss))
    return jnp.reshape(loss, (1,)).astype(jnp.float32)


# speedup vs baseline: 1.4325x; 1.4325x over previous
import jax
import jax.numpy as jnp
import numpy as np
from jax.experimental import pallas as pl
from jax.experimental.pallas import tpu as pltpu

N = 4 * 512 * 512
R, C = 8192, 128
GRID = 16
BR = R // GRID
PAD = np.float32(2.0)
ONE_SIGMA = np.float32(1.15)


def _part(gv, gi, pv, pi, base_mask):
    flag1 = gv / gi
    flag2 = gi / gv
    target = jnp.where(flag1 >= ONE_SIGMA, jnp.float32(1.0), jnp.float32(0.0))
    target = jnp.where(flag2 > ONE_SIGMA, jnp.float32(-1.0), target)
    diff = pv - pi
    nz = jnp.logical_and(target != 0.0, base_mask)
    z = jnp.logical_and(target == 0.0, base_mask)
    log_terms = jnp.log(1.0 + jnp.exp(-target * diff))
    s_log = jnp.sum(jnp.where(nz, log_terms, 0.0))
    c_nz = jnp.sum(nz.astype(jnp.float32))
    s_sq = jnp.sum(jnp.where(z, diff * diff, 0.0))
    c_z = jnp.sum(z.astype(jnp.float32))
    return s_log, c_nz, s_sq, c_z


def _rowshift2(x):
    return jnp.concatenate([x[:, 2:], x[:, :2]], axis=1)


def _thre_kernel(d_ref, out_ref):
    d = d_ref[...]
    valid = d > 0.0
    bits = pltpu.bitcast(d, jnp.int32)
    n_pos = jnp.sum(valid.astype(jnp.int32))

    q_index = jnp.float32(0.75) * (n_pos - 1)
    n_pos_f = (n_pos - 1).astype(jnp.float32)
    low = jnp.clip(jnp.floor(q_index), 0, n_pos_f)
    high = jnp.clip(jnp.ceil(q_index), 0, n_pos_f)
    high_weight = q_index - low
    low_weight = jnp.float32(1) - high_weight
    low_i = low.astype(jnp.int32)
    high_i = high.astype(jnp.int32)

    def body(_, lohi):
        lo, hi = lohi
        mid = (lo + hi) // 2
        cnt = jnp.sum(jnp.logical_and(valid, bits <= mid).astype(jnp.int32))
        take = cnt >= low_i + 1
        return (jnp.where(take, lo, mid + 1), jnp.where(take, mid, hi))

    lo_pat, _ = jax.lax.fori_loop(0, 30, body, (jnp.int32(0),
                                                jnp.int32(0x3F7FFFFF)))

    low_value = jnp.min(jnp.where(jnp.logical_and(valid, bits >= lo_pat),
                                  d, jnp.float32(2.0)))
    c_eq = jnp.sum(jnp.logical_and(valid, bits <= lo_pat).astype(jnp.int32))
    next_value = jnp.min(jnp.where(jnp.logical_and(valid, bits > lo_pat),
                                   d, jnp.float32(2.0)))
    high_value = jnp.where(high_i <= c_eq - 1, low_value, next_value)
    thre = low_value * low_weight + high_value * high_weight

    n_a = jnp.sum(jnp.logical_and(valid, d <= thre).astype(jnp.int32))
    n_b = jnp.sum((d > thre).astype(jnp.int32))

    slot = jax.lax.broadcasted_iota(jnp.int32, (1, 4), 1)
    vals = [thre, n_a.astype(jnp.float32), n_b.astype(jnp.float32),
            n_pos.astype(jnp.float32)]
    out_ref[...] = sum(v * (slot == k).astype(jnp.float32)
                       for k, v in enumerate(vals))


def _loss_kernel(s_ref, gn_ref, gf_ref, pn_ref, pf_ref, out_ref):
    step = pl.program_id(0)
    half = s_ref[0, 0]
    ms = s_ref[0, 1]

    gn = gn_ref[...]
    gf = gf_ref[...]
    pn = pn_ref[...]
    pf = pf_ref[...]

    row = jax.lax.broadcasted_iota(jnp.int32, (BR, C), 0) + step * BR
    lane = jax.lax.broadcasted_iota(jnp.int32, (BR, C), 1)
    p = row * C + lane
    mask12 = jnp.logical_and((p % 4) == 0, (p // 4) < half)
    mask3 = jnp.logical_and((p % 2) == 1, ((p - 1) // 2) < ms)

    r1 = _part(gn, _rowshift2(gn), pn, _rowshift2(pn), mask12)
    r2 = _part(gf, _rowshift2(gf), pf, _rowshift2(pf), mask12)
    r3 = _part(gn, gf, pn, pf, mask3)

    sums = [r1[k] + r2[k] + r3[k] for k in range(4)]
    slot = jax.lax.broadcasted_iota(jnp.int32, (1, 4), 1)
    vec = sum(sums[k] * (slot == k).astype(jnp.float32) for k in range(4))

    @pl.when(step == 0)
    def _init():
        out_ref[...] = vec

    @pl.when(step != 0)
    def _acc():
        out_ref[...] = out_ref[...] + vec


def kernel(pred_depth, gt_depth):
    pred = pred_depth.reshape(-1)
    depth = gt_depth.reshape(-1)
    valid = depth > 0

    stats = pl.pallas_call(
        _thre_kernel,
        out_shape=jax.ShapeDtypeStruct((1, 4), jnp.float32),
    )(depth.reshape(R, C))
    thre = stats[0, 0]
    n_a = stats[0, 1].astype(jnp.int32)
    n_b = stats[0, 2].astype(jnp.int32)
    n_pos = stats[0, 3].astype(jnp.int32)

    gn = jnp.sort(jnp.where(valid, depth, PAD))
    gf = jnp.roll(gn[::-1], -(N - n_pos))

    mask_A = jnp.logical_and(depth <= thre, valid)
    mask_B = depth > thre
    sp = jnp.sort(jnp.where(mask_B, -pred, jnp.where(mask_A, pred, PAD)))
    pf = -sp
    pn = jnp.roll(sp, -n_b)
    m = jnp.minimum(n_a, n_b)
    scalars = jnp.stack([m // 4, m // 2]).reshape(1, 2).astype(jnp.int32)

    sums = pl.pallas_call(
        _loss_kernel,
        grid=(GRID,),
        in_specs=[
            pl.BlockSpec((1, 2), lambda i: (0, 0)),
            pl.BlockSpec((BR, C), lambda i: (i, 0)),
            pl.BlockSpec((BR, C), lambda i: (i, 0)),
            pl.BlockSpec((BR, C), lambda i: (i, 0)),
            pl.BlockSpec((BR, C), lambda i: (i, 0)),
        ],
        out_specs=pl.BlockSpec((1, 4), lambda i: (0, 0)),
        out_shape=jax.ShapeDtypeStruct((1, 4), jnp.float32),
    )(scalars, gn.reshape(R, C), gf.reshape(R, C),
      pn.reshape(R, C), pf.reshape(R, C))

    log_loss = sums[0, 0] / sums[0, 1]
    squared_loss = sums[0, 2] / sums[0, 3]
    loss = jnp.where(jnp.isnan(log_loss), squared_loss,
                     jnp.where(jnp.isnan(squared_loss), log_loss,
                               log_loss + squared_loss))
    return jnp.reshape(loss, (1,)).astype(jnp.float32)
